# baseline (speedup 1.0000x reference)
import functools

import jax
import jax.numpy as jnp
from jax import lax
from jax.experimental import pallas as pl
from jax.experimental.pallas import tpu as pltpu
from jax.experimental.pallas import tpu_sc as plsc

_NW = 32


def _retile_table(weight):
    V, D = weight.shape
    CH = 320
    n_ch = V // CH
    base_ch = n_ch // _NW
    extra = n_ch - base_ch * _NW

    mesh = plsc.VectorSubcoreMesh(core_axis_name="c", subcore_axis_name="s")

    @functools.partial(
        pl.kernel,
        mesh=mesh,
        out_type=jax.ShapeDtypeStruct((V, D), jnp.float32),
        compiler_params=pltpu.CompilerParams(
            use_tc_tiling_on_sc=True, needs_layout_passes=False),
        scratch_types=[
            pltpu.VMEM((2, CH, D), jnp.float32),
            pltpu.SemaphoreType.DMA((2,)),
            pltpu.SemaphoreType.DMA((2,)),
        ],
    )
    def retile_kernel(w_hbm, out_hbm, buf, rsem, wsem):
        wid = lax.axis_index("s") * 2 + lax.axis_index("c")
        c0 = wid * base_ch

        def r_copy(c, h):
            return pltpu.make_async_copy(
                w_hbm.at[pl.ds(c * CH, CH), :], buf.at[h], rsem.at[h])

        def w_copy(c, h):
            return pltpu.make_async_copy(
                buf.at[h], out_hbm.at[pl.ds(c * CH, CH), :], wsem.at[h])

        xc = base_ch * _NW + wid
        n_my = base_ch + jnp.where(wid < extra, 1, 0)

        def cix(i):
            return jnp.where(i < base_ch, c0 + i, xc)

        r_copy(cix(0), 0).start()

        @pl.loop(0, n_my)
        def _(i):
            h = i & 1
            c = cix(i)
            r_copy(c, h).wait()
            w_copy(c, h).start()

            @pl.when(i + 1 < n_my)
            def _():
                @pl.when(i >= 1)
                def _():
                    w_copy(cix(i - 1), 1 - h).wait()

                r_copy(cix(i + 1), 1 - h).start()

        w_copy(cix(n_my - 2), (n_my - 2) & 1).wait()
        w_copy(cix(n_my - 1), (n_my - 1) & 1).wait()

    return retile_kernel(weight)


def kernel(token_ids, weight):
    B0, S = token_ids.shape
    V, D = weight.shape
    B = B0 * S
    idx = token_ids.reshape(B)
    weight = _retile_table(weight)
    BPT = B0 // _NW
    IPT = BPT * S
    NBT = BPT // 128
    NDT = D // 8
    SEC = 128 * 8 * NBT

    mesh = plsc.VectorSubcoreMesh(core_axis_name="c", subcore_axis_name="s")

    @functools.partial(
        pl.kernel,
        mesh=mesh,
        out_type=jax.ShapeDtypeStruct((S, B0 * D), jnp.float32),
        compiler_params=pltpu.CompilerParams(
            use_tc_tiling_on_sc=False, needs_layout_passes=False),
        scratch_types=[
            pltpu.VMEM((IPT,), jnp.int32),
            pltpu.VMEM((2, BPT), jnp.int32),
            pltpu.VMEM((2, BPT, D), jnp.float32),
            pltpu.VMEM((2, NDT * SEC), jnp.float32),
            pltpu.SemaphoreType.DMA((2,)),
            pltpu.SemaphoreType.DMA((2,)),
        ],
    )
    def gather_kernel(idx_hbm, table_hbm, z_hbm, idx_all, idx_s, chunk, zbuf,
                      gsem, wsem):
        wid = lax.axis_index("s") * 2 + lax.axis_index("c")
        ibase = wid * IPT
        bt0 = wid * NBT

        iota = lax.iota(jnp.int32, 16)

        def build_idx(s, h):
            @plsc.parallel_loop(0, BPT // 16, unroll=4)
            def _(k):
                pos = (k * 16 + iota) * S + s
                vals = plsc.load_gather(idx_all, [pos])
                idx_s.at[h][pl.ds(k * 16, 16)] = vals

        def g_copy(h):
            return pltpu.make_async_copy(
                table_hbm.at[idx_s.at[h]], chunk.at[h], gsem.at[h])

        def w_copy(s, h, dt):
            return pltpu.make_async_copy(
                zbuf.at[h].at[pl.ds(dt * SEC, SEC)],
                z_hbm.at[s].at[pl.ds((dt * 128 + bt0) * 1024, SEC)],
                wsem.at[h],
            )

        def out_start(s, h):
            for dt in range(NDT):
                w_copy(s, h, dt).start()

        def out_wait(s, h):
            for dt in range(NDT):
                w_copy(s, h, dt).wait()

        def transpose(h):
            @plsc.parallel_loop(0, NBT * 8, unroll=4)
            def _(k):
                btl = k >> 3
                blg = k & 7
                rows = btl * 128 + blg * 16 + iota
                off_b = btl * 1024 + blg * 16
                for dt in range(NDT):
                    for dr in range(8):
                        col = jnp.full((16,), dt * 8 + dr, dtype=jnp.int32)
                        v = plsc.load_gather(chunk.at[h], [rows, col])
                        zbuf.at[h][pl.ds(off_b + dt * SEC + dr * 128, 16)] = v

        pltpu.sync_copy(idx_hbm.at[pl.ds(ibase, IPT)], idx_all)

        def step(s, h, prefetch, outwait):
            if prefetch:
                build_idx(s + 1, 1 - h)
                g_copy(1 - h).start()
            g_copy(h).wait()
            if outwait:
                out_wait(s - 2, h)
            transpose(h)
            out_start(s, h)

        build_idx(0, 0)
        g_copy(0).start()
        step(0, 0, True, False)
        step(1, 1, True, False)

        @pl.loop(2, S - 2, step=2)
        def _(g):
            step(g, 0, True, True)
            step(g + 1, 1, True, True)

        step(S - 2, 0, True, True)
        step(S - 1, 1, False, True)
        out_wait(S - 2, 0)
        out_wait(S - 1, 1)

    z = gather_kernel(idx, weight)
    out = (z.reshape(S, NDT, 128, 8, 128)
            .transpose(2, 4, 0, 1, 3)
            .reshape(B0, S, D))
    return out


# speedup vs baseline: 1.5282x; 1.5282x over previous
import functools

import jax
import jax.numpy as jnp
from jax import lax
from jax.experimental import pallas as pl
from jax.experimental.pallas import tpu as pltpu
from jax.experimental.pallas import tpu_sc as plsc

_NW = 32


def kernel(token_ids, weight):
    B0, S = token_ids.shape
    V, D = weight.shape
    B = B0 * S
    idx = token_ids.reshape(B)
    BPT = B0 // _NW
    IPT = BPT * S
    NBT = BPT // 128
    NDT = D // 8
    SEC = 128 * 8 * NBT

    mesh = plsc.VectorSubcoreMesh(core_axis_name="c", subcore_axis_name="s")

    @functools.partial(
        pl.kernel,
        mesh=mesh,
        out_type=jax.ShapeDtypeStruct((S, B0 * D), jnp.float32),
        compiler_params=pltpu.CompilerParams(
            use_tc_tiling_on_sc=False, needs_layout_passes=False),
        scratch_types=[
            pltpu.VMEM((IPT,), jnp.int32),
            pltpu.VMEM((2, BPT), jnp.int32),
            pltpu.VMEM((2, BPT, D), jnp.float32),
            pltpu.VMEM((2, NDT * SEC), jnp.float32),
            pltpu.SemaphoreType.DMA((2,)),
            pltpu.SemaphoreType.DMA((2,)),
        ],
    )
    def gather_kernel(idx_hbm, table_hbm, z_hbm, idx_all, idx_s, chunk, zbuf,
                      gsem, wsem):
        wid = lax.axis_index("s") * 2 + lax.axis_index("c")
        ibase = wid * IPT
        bt0 = wid * NBT

        iota = lax.iota(jnp.int32, 16)

        def build_idx(s, h):
            @plsc.parallel_loop(0, BPT // 16, unroll=4)
            def _(k):
                pos = (k * 16 + iota) * S + s
                vals = plsc.load_gather(idx_all, [pos])
                idx_s.at[h][pl.ds(k * 16, 16)] = vals

        def g_copies(h):
            half = BPT // 2
            return [
                pltpu.make_async_copy(
                    table_hbm.at[idx_s.at[h].at[pl.ds(q * half, half)]],
                    chunk.at[h].at[pl.ds(q * half, half), :],
                    gsem.at[h])
                for q in range(2)
            ]

        def g_copy_start(h):
            for cp in g_copies(h):
                cp.start()

        def g_copy_wait(h):
            for cp in g_copies(h):
                cp.wait()

        def w_copy(s, h, dt):
            return pltpu.make_async_copy(
                zbuf.at[h].at[pl.ds(dt * SEC, SEC)],
                z_hbm.at[s].at[pl.ds((dt * 128 + bt0) * 1024, SEC)],
                wsem.at[h],
            )

        def out_start(s, h):
            for dt in range(NDT):
                w_copy(s, h, dt).start()

        def out_wait(s, h):
            for dt in range(NDT):
                w_copy(s, h, dt).wait()

        def transpose(h):
            @plsc.parallel_loop(0, NBT * 8, unroll=4)
            def _(k):
                btl = k >> 3
                blg = k & 7
                rows = btl * 128 + blg * 16 + iota
                off_b = btl * 1024 + blg * 16
                for dt in range(NDT):
                    for dr in range(8):
                        col = jnp.full((16,), dt * 8 + dr, dtype=jnp.int32)
                        v = plsc.load_gather(chunk.at[h], [rows, col])
                        zbuf.at[h][pl.ds(off_b + dt * SEC + dr * 128, 16)] = v

        pltpu.sync_copy(idx_hbm.at[pl.ds(ibase, IPT)], idx_all)

        def step(s, h, prefetch, outwait):
            if prefetch:
                build_idx(s + 1, 1 - h)
                g_copy_start(1 - h)
            g_copy_wait(h)
            if outwait:
                out_wait(s - 2, h)
            transpose(h)
            out_start(s, h)

        build_idx(0, 0)
        g_copy_start(0)
        step(0, 0, True, False)
        step(1, 1, True, False)

        @pl.loop(2, S - 2, step=2)
        def _(g):
            step(g, 0, True, True)
            step(g + 1, 1, True, True)

        step(S - 2, 0, True, True)
        step(S - 1, 1, False, True)
        out_wait(S - 2, 0)
        out_wait(S - 1, 1)

    z = gather_kernel(idx, weight)
    out = (z.reshape(S, NDT, 128, 8, 128)
            .transpose(2, 4, 0, 1, 3)
            .reshape(B0, S, D))
    return out
